# Initial kernel scaffold; baseline (speedup 1.0000x reference)
import jax
import jax.numpy as jnp
from jax.experimental import pallas as pl


def kernel(input_ids, token_embedding, position_embedding, special_token_embedding):
    raise NotImplementedError("write your pallas kernel here")


# speedup vs baseline: 4.4652x; 4.4652x over previous
import functools

import jax
import jax.numpy as jnp
from jax import lax
from jax.experimental import pallas as pl
from jax.experimental.pallas import tpu as pltpu
from jax.experimental.pallas import tpu_sc as plsc

_L = 8192
_D = 768
_NW = 32
_RPW = _L // _NW
_W = 64
_NCH = _RPW // _W
_LANES = 16


def _sc_body(tok_hbm, pos_hbm, tokidx_hbm, posidx_hbm, spec_hbm, o_hbm,
             idx_v, pidx_v, tok_v, pos_v, spec_v, sem0, sem1):
    c_id = lax.axis_index("c")
    s_id = lax.axis_index("s")
    wid = s_id * 2 + c_id
    base = wid * _RPW

    pltpu.sync_copy(tokidx_hbm.at[wid], idx_v)
    pltpu.sync_copy(posidx_hbm.at[wid], pidx_v)
    pltpu.sync_copy(spec_hbm, spec_v)

    @pl.loop(0, _NCH)
    def _chunk(ch):
        cp0 = pltpu.async_copy(tok_hbm.at[idx_v.at[ch]], tok_v, sem0)
        cp1 = pltpu.async_copy(pos_hbm.at[pidx_v.at[ch]], pos_v, sem1)
        cp0.wait()
        cp1.wait()

        @pl.loop(0, _W)
        def _row(r):
            for c in range(0, _D, _LANES):
                tok_v[r, pl.ds(c, _LANES)] += pos_v[r, pl.ds(c, _LANES)]

        @pl.when(jnp.logical_and(wid == 0, ch == 0))
        def _special():
            for c in range(0, _D, _LANES):
                tok_v[1, pl.ds(c, _LANES)] = spec_v[pl.ds(c, _LANES)]

        pltpu.sync_copy(tok_v, o_hbm.at[pl.ds(base + ch * _W, _W)])


@jax.jit
def _embed(token_embedding, position_embedding, tok_idx, pos_idx, spec):
    mesh = plsc.VectorSubcoreMesh(core_axis_name="c", subcore_axis_name="s")
    run = pl.kernel(
        _sc_body,
        out_type=jax.ShapeDtypeStruct((_L, _D), jnp.float32),
        mesh=mesh,
        scratch_types=[
            pltpu.VMEM((_NCH, _W), jnp.int32),
            pltpu.VMEM((_NCH, _W), jnp.int32),
            pltpu.VMEM((_W, _D), jnp.float32),
            pltpu.VMEM((_W, _D), jnp.float32),
            pltpu.VMEM((_D,), jnp.float32),
            pltpu.SemaphoreType.DMA,
            pltpu.SemaphoreType.DMA,
        ],
    )
    return run(token_embedding, position_embedding, tok_idx, pos_idx, spec)


def kernel(input_ids, token_embedding, position_embedding, special_token_embedding):
    ids = input_ids[0]
    tok_idx = jnp.concatenate(
        [ids[1:2], jnp.zeros((1,), jnp.int32), ids[2:]]
    )
    j = jnp.arange(_L, dtype=jnp.int32)
    pos_idx = jnp.maximum(j - 1, 0)
    tok_idx = tok_idx.reshape(_NW, _NCH, _W)
    pos_idx = pos_idx.reshape(_NW, _NCH, _W)
    spec = special_token_embedding.reshape(_D)
    out = _embed(token_embedding, position_embedding, tok_idx, pos_idx, spec)
    return out[None]


# speedup vs baseline: 5.4072x; 1.2110x over previous
import jax
import jax.numpy as jnp
from jax import lax
from jax.experimental import pallas as pl
from jax.experimental.pallas import tpu as pltpu
from jax.experimental.pallas import tpu_sc as plsc

_L = 8192
_D = 768
_NW = 32
_RPW = _L // _NW
_W = 32
_NCH = _RPW // _W
_LANES = 16
_NTB = 3
_NPB = 2


def _sc_body(tok_hbm, pos_hbm, tokidx_hbm, posidx_hbm, spec_hbm, o_hbm,
             idx_v, pidx_v, spec_v,
             tb0, tb1, tb2, pb0, pb1,
             gs0, gs1, gs2, ws0, ws1, ws2):
    tbufs = (tb0, tb1, tb2)
    pbufs = (pb0, pb1)
    gsems = (gs0, gs1, gs2)
    wsems = (ws0, ws1, ws2)

    c_id = lax.axis_index("c")
    s_id = lax.axis_index("s")
    wid = s_id * 2 + c_id
    base = wid * _RPW

    pltpu.sync_copy(tokidx_hbm.at[wid], idx_v)
    pltpu.sync_copy(posidx_hbm.at[wid], pidx_v)
    pltpu.sync_copy(spec_hbm, spec_v)

    def start_gathers(c):
        tcp = pltpu.async_copy(tok_hbm.at[idx_v.at[c]], tbufs[c % _NTB],
                               gsems[c % _NTB])
        pcp = pltpu.async_copy(pos_hbm.at[pidx_v.at[c]], pbufs[c % _NPB],
                               gsems[c % _NTB])
        return (tcp, pcp)

    gathers = {}
    writes = {}
    gathers[0] = start_gathers(0)
    gathers[1] = start_gathers(1)

    for c in range(_NCH):
        b = c % _NTB
        tok_v = tbufs[b]
        pos_v = pbufs[c % _NPB]
        gathers[c][0].wait()
        gathers[c][1].wait()

        @pl.loop(0, _W)
        def _row(r):
            for col in range(0, _D, _LANES):
                tok_v[r, pl.ds(col, _LANES)] += pos_v[r, pl.ds(col, _LANES)]

        if c == 0:
            @pl.when(wid == 0)
            def _special():
                for col in range(0, _D, _LANES):
                    tok_v[1, pl.ds(col, _LANES)] = spec_v[pl.ds(col, _LANES)]

        writes[c] = pltpu.async_copy(
            tok_v, o_hbm.at[pl.ds(base + c * _W, _W)], wsems[b])

        nxt = c + 2
        if nxt < _NCH:
            if nxt - _NTB >= 0:
                writes[nxt - _NTB].wait()
            gathers[nxt] = start_gathers(nxt)

    for c in range(_NCH - _NTB, _NCH):
        writes[c].wait()


@jax.jit
def _embed(token_embedding, position_embedding, tok_idx, pos_idx, spec):
    mesh = plsc.VectorSubcoreMesh(core_axis_name="c", subcore_axis_name="s")
    run = pl.kernel(
        _sc_body,
        out_type=jax.ShapeDtypeStruct((_L, _D), jnp.float32),
        mesh=mesh,
        scratch_types=(
            [
                pltpu.VMEM((_NCH, _W), jnp.int32),
                pltpu.VMEM((_NCH, _W), jnp.int32),
                pltpu.VMEM((_D,), jnp.float32),
            ]
            + [pltpu.VMEM((_W, _D), jnp.float32)] * (_NTB + _NPB)
            + [pltpu.SemaphoreType.DMA] * 6
        ),
    )
    return run(token_embedding, position_embedding, tok_idx, pos_idx, spec)


def kernel(input_ids, token_embedding, position_embedding, special_token_embedding):
    ids = input_ids[0]
    tok_idx = jnp.concatenate(
        [ids[1:2], jnp.zeros((1,), jnp.int32), ids[2:]]
    )
    j = jnp.arange(_L, dtype=jnp.int32)
    pos_idx = jnp.maximum(j - 1, 0)
    tok_idx = tok_idx.reshape(_NW, _NCH, _W)
    pos_idx = pos_idx.reshape(_NW, _NCH, _W)
    spec = special_token_embedding.reshape(_D)
    out = _embed(token_embedding, position_embedding, tok_idx, pos_idx, spec)
    return out[None]
